# baseline (speedup 1.0000x reference)
import functools

import jax
import jax.numpy as jnp
from jax.experimental import pallas as pl
from jax.experimental.pallas import tpu as pltpu
from jax.experimental.pallas import tpu_sc as plsc

_N = 10000
_E = 320000
_D = 128
_NC = 2
_NS = 16
_NW = _NC * _NS
_K = 128
_CH = 80
_EPW = _CH * _K
_EP = _NW * _EPW
_NBUF = 2
_NA = _N + 16
_RPT = 624
_REM_BASE = _NS * _RPT
_REM = _N - _REM_BASE

_BLK = 1000
_G = _N // _BLK

_sc_mesh = functools.partial(
    plsc.VectorSubcoreMesh, core_axis_name="c", subcore_axis_name="s"
)


def _zero_acc(zero_hbm, acc, s):
    pltpu.sync_copy(zero_hbm.at[pl.ds(s * _RPT, _RPT)],
                    acc.at[pl.ds(s * _RPT, _RPT)])

    @pl.when(s == 0)
    def _():
        pltpu.sync_copy(zero_hbm.at[pl.ds(_REM_BASE, _REM)],
                        acc.at[pl.ds(_REM_BASE, _REM)])


def _drain_acc(acc, out_hbm, c, s):
    pltpu.sync_copy(acc.at[pl.ds(s * _RPT, _RPT)],
                    out_hbm.at[c, pl.ds(s * _RPT, _RPT)])

    @pl.when(s == 0)
    def _():
        pltpu.sync_copy(acc.at[pl.ds(_REM_BASE, _REM)],
                        out_hbm.at[c, pl.ds(_REM_BASE, _REM)])


def _sc_degree(dst3, ones_rows, zeros):
    @functools.partial(
        pl.kernel,
        out_type=jax.ShapeDtypeStruct((_NC, _N, _D), jnp.float32),
        mesh=_sc_mesh(),
        scratch_types=[
            pltpu.VMEM((_CH, _K), jnp.int32),
            pltpu.VMEM((_K, _D), jnp.float32),
            pltpu.VMEM_SHARED((_NA, _D), jnp.float32),
            pltpu.SemaphoreType.DMA((_NBUF,)),
        ],
    )
    def k(dst_hbm, ones_hbm, zero_hbm, out_hbm, didx, ones_v, acc, sems):
        c = jax.lax.axis_index("c")
        s = jax.lax.axis_index("s")
        wid = c * _NS + s
        pltpu.sync_copy(ones_hbm, ones_v)
        pltpu.sync_copy(dst_hbm.at[wid], didx)
        _zero_acc(zero_hbm, acc, s)
        plsc.subcore_barrier()

        for b in range(_NBUF):
            pltpu.async_copy(ones_v, acc.at[didx.at[b]], add=True,
                             sem=sems.at[b])

        @pl.loop(0, _CH - _NBUF, step=_NBUF)
        def _(j):
            for b in range(_NBUF):
                pltpu.make_async_copy(ones_v, acc.at[didx.at[0]],
                                      sems.at[b]).wait()
                pltpu.async_copy(ones_v, acc.at[didx.at[j + _NBUF + b]],
                                 add=True, sem=sems.at[b])

        for b in range(_NBUF):
            pltpu.make_async_copy(ones_v, acc.at[didx.at[0]],
                                  sems.at[b]).wait()

        plsc.subcore_barrier()
        _drain_acc(acc, out_hbm, c, s)

    return k(dst3, ones_rows, zeros)


def _sc_propagate(hs, src3, dst3, zeros):
    @functools.partial(
        pl.kernel,
        out_type=jax.ShapeDtypeStruct((_NC, _N, _D), jnp.float32),
        mesh=_sc_mesh(),
        scratch_types=[
            [pltpu.VMEM((_K,), jnp.int32) for _ in range(2 * _NBUF)],
            pltpu.VMEM((_CH, _K), jnp.int32),
            [pltpu.VMEM((_K, _D), jnp.float32) for _ in range(_NBUF)],
            pltpu.VMEM_SHARED((_NA, _D), jnp.float32),
            pltpu.SemaphoreType.DMA((_NBUF,)),
            pltpu.SemaphoreType.DMA((2 * _NBUF,)),
        ],
    )
    def k(hs_hbm, src_hbm, dst_hbm, zero_hbm, out_hbm,
          sidx, didx, rows, acc, gsem, isem):
        c = jax.lax.axis_index("c")
        s = jax.lax.axis_index("s")
        wid = c * _NS + s
        pltpu.sync_copy(dst_hbm.at[wid], didx)
        _zero_acc(zero_hbm, acc, s)
        plsc.subcore_barrier()

        def fire_idx(chunk, i):
            pltpu.async_copy(src_hbm.at[wid, chunk], sidx[i], isem.at[i])

        def wait_idx(i):
            pltpu.make_async_copy(src_hbm.at[wid, 0], sidx[i],
                                  isem.at[i]).wait()

        def fire_gather(i, b):
            pltpu.async_copy(hs_hbm.at[sidx[i]], rows[b], gsem.at[b])

        def wait_gather(i, b):
            pltpu.make_async_copy(hs_hbm.at[pl.ds(0, _K)], rows[b],
                                  gsem.at[b]).wait()

        for i in range(2 * _NBUF):
            fire_idx(i, i)
        for b in range(_NBUF):
            wait_idx(b)
            fire_gather(b, b)

        @pl.loop(0, _CH, step=2 * _NBUF)
        def _(j):
            for u in range(2 * _NBUF):
                b = u % _NBUF
                i = u
                cch = j + u
                wait_gather(i, b)
                pltpu.sync_copy(rows[b], acc.at[didx.at[cch]], add=True)

                @pl.when(cch + 2 * _NBUF < _CH)
                def _():
                    fire_idx(cch + 2 * _NBUF, i)

                @pl.when(cch + _NBUF < _CH)
                def _():
                    i4 = (u + _NBUF) % (2 * _NBUF)
                    wait_idx(i4)
                    fire_gather(i4, b)

        plsc.subcore_barrier()
        _drain_acc(acc, out_hbm, c, s)

    return k(hs, src3, dst3, zeros)


def _dis_block(degp):
    deg = 1.0 + degp[0, :, 0:1] + degp[1, :, 0:1]
    return jax.lax.rsqrt(deg), deg


_row_spec = pl.BlockSpec((_BLK, _D), lambda i: (i, 0))
_p_spec = pl.BlockSpec((_NC, _BLK, _D), lambda i: (0, i, 0))
_deg_spec = pl.BlockSpec((_NC, _BLK, _D), lambda i: (0, i, 0))
_w_spec = pl.BlockSpec((_D, _D), lambda i: (0, 0))
_b_spec = pl.BlockSpec((1, _D), lambda i: (0, 0))


def _tc_mm(x, W1):
    def body(x_ref, w_ref, h_ref):
        h_ref[...] = jnp.dot(x_ref[...], w_ref[...],
                             preferred_element_type=jnp.float32)

    return pl.pallas_call(
        body,
        grid=(_G,),
        in_specs=[_row_spec, _w_spec],
        out_specs=_row_spec,
        out_shape=jax.ShapeDtypeStruct((_N, _D), jnp.float32),
    )(x, W1)


def _tc_scale(h, degp):
    def body(h_ref, degp_ref, hs_ref):
        dis, _ = _dis_block(degp_ref[...])
        hs_ref[...] = h_ref[...] * dis

    return pl.pallas_call(
        body,
        grid=(_G,),
        in_specs=[_row_spec, _deg_spec],
        out_specs=_row_spec,
        out_shape=jax.ShapeDtypeStruct((_N, _D), jnp.float32),
    )(h, degp)


def _tc_mid(p, h, degp, b, res, Wn):
    have_res = res is not None

    def body(*refs):
        if have_res:
            p_ref, h_ref, degp_ref, b_ref, res_ref, w_ref, a_ref, hn_ref, hs_ref = refs
        else:
            p_ref, h_ref, degp_ref, b_ref, w_ref, a_ref, hn_ref, hs_ref = refs
        dis, deg = _dis_block(degp_ref[...])
        agg = p_ref[0] + p_ref[1]
        a = dis * agg + h_ref[...] / deg + b_ref[...]
        if have_res:
            a = a + res_ref[...]
        a = jnp.maximum(a, 0.0)
        hn = jnp.dot(a, w_ref[...], preferred_element_type=jnp.float32)
        a_ref[...] = a
        hn_ref[...] = hn
        hs_ref[...] = hn * dis

    in_specs = [_p_spec, _row_spec, _deg_spec, _b_spec]
    args = [p, h, degp, b.reshape(1, _D)]
    if have_res:
        in_specs.append(_row_spec)
        args.append(res)
    in_specs.append(_w_spec)
    args.append(Wn)
    return pl.pallas_call(
        body,
        grid=(_G,),
        in_specs=in_specs,
        out_specs=[_row_spec, _row_spec, _row_spec],
        out_shape=[
            jax.ShapeDtypeStruct((_N, _D), jnp.float32),
            jax.ShapeDtypeStruct((_N, _D), jnp.float32),
            jax.ShapeDtypeStruct((_N, _D), jnp.float32),
        ],
    )(*args)


def _tc_final(p, h, degp, b):
    def body(p_ref, h_ref, degp_ref, b_ref, o_ref):
        dis, deg = _dis_block(degp_ref[...])
        a = dis * (p_ref[0] + p_ref[1]) + h_ref[...] / deg + b_ref[...]
        o_ref[...] = jax.nn.sigmoid(a)

    return pl.pallas_call(
        body,
        grid=(_G,),
        in_specs=[_p_spec, _row_spec, _deg_spec, _b_spec],
        out_specs=_row_spec,
        out_shape=jax.ShapeDtypeStruct((_N, _D), jnp.float32),
    )(p, h, degp, b.reshape(1, _D))


def kernel(x, edge_index, W1, b1, W2, b2, W3, b3):
    src = edge_index[0]
    dst = edge_index[1]
    pad = _EP - _E
    src3 = jnp.concatenate([src, jnp.zeros((pad,), jnp.int32)]
                           ).reshape(_NW, _CH, _K)
    dst3 = jnp.concatenate([dst, jnp.full((pad,), _N, jnp.int32)]
                           ).reshape(_NW, _CH, _K)
    zeros = jnp.zeros((_N, _D), jnp.float32)
    ones_rows = jnp.ones((_K, _D), jnp.float32)

    degp = _sc_degree(dst3, ones_rows, zeros)
    h1 = _tc_mm(x, W1)
    hs1 = _tc_scale(h1, degp)
    p1 = _sc_propagate(hs1, src3, dst3, zeros)

    act1, h2, hs2 = _tc_mid(p1, h1, degp, b1, None, W2)
    p2 = _sc_propagate(hs2, src3, dst3, zeros)

    _, h3, hs3 = _tc_mid(p2, h2, degp, b2, act1, W3)
    p3 = _sc_propagate(hs3, src3, dst3, zeros)

    return _tc_final(p3, h3, degp, b3)
